# Initial kernel scaffold; baseline (speedup 1.0000x reference)
import jax
import jax.numpy as jnp
from jax.experimental import pallas as pl


def kernel(x, params, edge_index):
    raise NotImplementedError("write your pallas kernel here")


# speedup vs baseline: 13.5275x; 13.5275x over previous
import functools

import jax
import jax.numpy as jnp
from jax import lax
from jax.experimental import pallas as pl
from jax.experimental.pallas import tpu as pltpu
from jax.experimental.pallas import tpu_sc as plsc

_NC = 2
_NS = 16
_NW = _NC * _NS
_EB = 128
_ROWS = 1000


def _make_propagate(n_pad, C, K):
    rpt = n_pad // _NS
    mesh = plsc.VectorSubcoreMesh(
        core_axis_name="c", subcore_axis_name="s",
        num_cores=_NC, num_subcores=_NS)

    @functools.partial(
        pl.kernel,
        out_type=jax.ShapeDtypeStruct((_NC, n_pad, C), jnp.float32),
        mesh=mesh,
        compiler_params=pltpu.CompilerParams(use_tc_tiling_on_sc=False),
        scratch_types=[
            pltpu.VMEM((K, _EB), jnp.int32),
            pltpu.VMEM((K, _EB), jnp.int32),
            pltpu.VMEM((_EB, C), jnp.float32),
            pltpu.VMEM((rpt, C), jnp.float32),
            pltpu.VMEM_SHARED((n_pad, C), jnp.float32),
            pltpu.SemaphoreType.DMA,
        ],
    )
    def prop(t_hbm, src_hbm, dst_hbm, zeros_hbm, out_hbm,
             src_v, dst_v, buf, row_v, acc, sem):
        c = lax.axis_index("c")
        s = lax.axis_index("s")
        w = s * _NC + c
        pltpu.sync_copy(src_hbm.at[w], src_v)
        pltpu.sync_copy(dst_hbm.at[w], dst_v)
        r0 = s * rpt
        pltpu.sync_copy(zeros_hbm, row_v)
        pltpu.sync_copy(row_v, acc.at[pl.ds(r0, rpt)])
        plsc.subcore_barrier()

        def body(j, carry):
            pltpu.async_copy(t_hbm.at[src_v.at[j]], buf, sem).wait()
            pltpu.sync_copy(buf, acc.at[dst_v.at[j]], add=True)
            return carry

        lax.fori_loop(0, K, body, 0)
        plsc.subcore_barrier()
        pltpu.sync_copy(acc.at[pl.ds(r0, rpt)], row_v)
        pltpu.sync_copy(row_v, out_hbm.at[c, pl.ds(r0, rpt)])

    return prop


def _mm_scale(h, wc, dinv):
    n, d = h.shape
    hdim = wc.shape[1]

    def kern(h_ref, w_ref, d_ref, o_ref):
        p = jnp.dot(h_ref[...], w_ref[...], preferred_element_type=jnp.float32)
        col = lax.broadcasted_iota(jnp.int32, (_ROWS, hdim), 1)
        o_ref[...] = p * jnp.where(col < 32, 1.0, d_ref[...])

    return pl.pallas_call(
        kern,
        grid=(n // _ROWS,),
        in_specs=[
            pl.BlockSpec((_ROWS, d), lambda i: (i, 0)),
            pl.BlockSpec((d, hdim), lambda i: (0, 0)),
            pl.BlockSpec((_ROWS, 1), lambda i: (i, 0)),
        ],
        out_specs=pl.BlockSpec((_ROWS, hdim), lambda i: (i, 0)),
        out_shape=jax.ShapeDtypeStruct((n, hdim), jnp.float32),
    )(h, wc, dinv)


def _mid(p0, p1, t, dinv):
    n, cdim = t.shape

    def kern(a_ref, b_ref, t_ref, d_ref, o_ref):
        d = d_ref[...]
        u = (a_ref[...] + b_ref[...] + t_ref[...]) * d
        col = lax.broadcasted_iota(jnp.int32, (_ROWS, cdim), 1)
        o_ref[...] = u * jnp.where(col < 32, 1.0, d)

    spec = pl.BlockSpec((_ROWS, cdim), lambda i: (i, 0))
    return pl.pallas_call(
        kern,
        grid=(n // _ROWS,),
        in_specs=[spec, spec, spec, pl.BlockSpec((_ROWS, 1), lambda i: (i, 0))],
        out_specs=spec,
        out_shape=jax.ShapeDtypeStruct((n, cdim), jnp.float32),
    )(p0, p1, t, dinv)


def _fin(q0, q1, t2, p_first, u_first, bc, dinv):
    n, cdim = t2.shape

    def kern(q0r, q1r, t2r, pr, ur, br, dr, o_ref):
        v = (q0r[...] + q1r[...] + t2r[...]) * dr[...]
        cat = jnp.concatenate([pr[...], ur[...], v], axis=1)
        o_ref[...] = jnp.tanh(cat + br[...])

    spec32 = pl.BlockSpec((_ROWS, cdim), lambda i: (i, 0))
    return pl.pallas_call(
        kern,
        grid=(n // _ROWS,),
        in_specs=[
            spec32, spec32, spec32, spec32, spec32,
            pl.BlockSpec((1, 3 * cdim), lambda i: (0, 0)),
            pl.BlockSpec((_ROWS, 1), lambda i: (i, 0)),
        ],
        out_specs=pl.BlockSpec((_ROWS, 3 * cdim), lambda i: (i, 0)),
        out_shape=jax.ShapeDtypeStruct((n, 3 * cdim), jnp.float32),
    )(q0, q1, t2, p_first, u_first, bc, dinv)


def _dinv_from_counts(c0, c1):
    n = c0.shape[0]

    def kern(a_ref, b_ref, o_ref):
        o_ref[...] = lax.rsqrt(a_ref[...] + b_ref[...] + 1.0)

    spec = pl.BlockSpec((_ROWS, 1), lambda i: (i, 0))
    return pl.pallas_call(
        kern,
        grid=(n // _ROWS,),
        in_specs=[spec, spec],
        out_specs=spec,
        out_shape=jax.ShapeDtypeStruct((n, 1), jnp.float32),
    )(c0, c1)


def kernel(x, params, edge_index):
    n, _ = x.shape
    e = edge_index.shape[1]
    n_pad = -(-(n + 1) // (_NS * 8)) * (_NS * 8)
    k = -(-e // (_NW * _EB))
    e_pad = k * _NW * _EB
    pad_rows = n_pad - n

    src = jnp.concatenate(
        [edge_index[0], jnp.zeros((e_pad - e,), jnp.int32)]).reshape(_NW, k, _EB)
    dst = jnp.concatenate(
        [edge_index[1], jnp.full((e_pad - e,), n, jnp.int32)]).reshape(_NW, k, _EB)

    rpt = n_pad // _NS
    z16 = jnp.zeros((rpt, 16), jnp.float32)
    z32 = jnp.zeros((rpt, 32), jnp.float32)
    z64 = jnp.zeros((rpt, 64), jnp.float32)
    prop16 = _make_propagate(n_pad, 16, k)
    prop32 = _make_propagate(n_pad, 32, k)
    prop64 = _make_propagate(n_pad, 64, k)

    cnt = prop16(jnp.ones((n_pad, 16), jnp.float32), src, dst, z16)
    dinv = _dinv_from_counts(cnt[0, :n, :1], cnt[1, :n, :1])

    h = x
    for lname in ("1", "2", "3"):
        wc = jnp.concatenate(params["W" + lname], axis=1)
        bc = jnp.concatenate(params["b" + lname]).reshape(1, -1)
        P = _mm_scale(h, wc, dinv)
        t = P[:, 32:]
        t_pad = jnp.concatenate([t, jnp.zeros((pad_rows, 64), jnp.float32)], 0)
        pp = prop64(t_pad, src, dst, z64)
        U = _mid(pp[0, :n], pp[1, :n], t, dinv)
        t2 = U[:, 32:]
        t2_pad = jnp.concatenate([t2, jnp.zeros((pad_rows, 32), jnp.float32)], 0)
        qq = prop32(t2_pad, src, dst, z32)
        h = _fin(qq[0, :n], qq[1, :n], t2, P[:, :32], U[:, :32], bc, dinv)
    return h
